# baseline (speedup 1.0000x reference)
import jax
import jax.numpy as jnp
from jax.experimental import pallas as pl
from jax.experimental.pallas import tpu as pltpu

D_MODEL = 2048
NUM_EXPERTS = 64
TOKENS = 16384
BLOCK = 512


def _router_block(x_ref, w_ref, b_ref, probs_ref, idx_ref):
    x = x_ref[...]
    w = w_ref[...]
    logits = jnp.dot(x, w, preferred_element_type=jnp.float32) + b_ref[...]
    iota = jax.lax.broadcasted_iota(jnp.int32, logits.shape, 1)

    m1 = jnp.max(logits, axis=1, keepdims=True)
    i1 = jnp.min(jnp.where(logits == m1, iota, NUM_EXPERTS), axis=1,
                 keepdims=True)
    masked = jnp.where(iota == i1, -jnp.inf, logits)
    m2 = jnp.max(masked, axis=1, keepdims=True)
    i2 = jnp.min(jnp.where(masked == m2, iota, NUM_EXPERTS), axis=1,
                 keepdims=True)

    e2 = jnp.exp(m2 - m1)
    p1 = 1.0 / (1.0 + e2)
    p2 = 1.0 - p1

    probs_ref[0] = jnp.concatenate([p1, p2], axis=1)
    idx_ref[0] = jnp.concatenate([i1, i2], axis=1)


def kernel(x, W, b):
    grid = TOKENS // BLOCK
    probs, idx = pl.pallas_call(
        _router_block,
        grid=(grid,),
        compiler_params=pltpu.CompilerParams(
            dimension_semantics=("parallel",),
        ),
        in_specs=[
            pl.BlockSpec((BLOCK, D_MODEL), lambda i: (i, 0)),
            pl.BlockSpec((D_MODEL, NUM_EXPERTS), lambda i: (0, 0)),
            pl.BlockSpec((1, NUM_EXPERTS), lambda i: (0, 0)),
        ],
        out_specs=[
            pl.BlockSpec((1, BLOCK, 2), lambda i: (i, 0, 0)),
            pl.BlockSpec((1, BLOCK, 2), lambda i: (i, 0, 0)),
        ],
        out_shape=[
            jax.ShapeDtypeStruct((grid, BLOCK, 2), jnp.float32),
            jax.ShapeDtypeStruct((grid, BLOCK, 2), jnp.int32),
        ],
    )(x, W.astype(jnp.float32), b.reshape(1, NUM_EXPERTS))
    return probs.reshape(TOKENS, 2), idx.reshape(TOKENS, 2)


# speedup vs baseline: 1.1701x; 1.1701x over previous
import jax
import jax.numpy as jnp
from jax.experimental import pallas as pl
from jax.experimental.pallas import tpu as pltpu

D_MODEL = 2048
NUM_EXPERTS = 64
TOKENS = 16384
BLOCK = 1024


def _router_block(x_ref, w_ref, b_ref, probs_ref, idx_ref):
    x = x_ref[...]
    w = w_ref[...]
    logits = jnp.dot(x, w, preferred_element_type=jnp.float32) + b_ref[...]
    iota = jax.lax.broadcasted_iota(jnp.int32, logits.shape, 1)

    m1 = jnp.max(logits, axis=1, keepdims=True)
    i1 = jnp.min(jnp.where(logits == m1, iota, NUM_EXPERTS), axis=1,
                 keepdims=True)
    masked = jnp.where(iota == i1, -jnp.inf, logits)
    m2 = jnp.max(masked, axis=1, keepdims=True)
    i2 = jnp.min(jnp.where(masked == m2, iota, NUM_EXPERTS), axis=1,
                 keepdims=True)

    e2 = jnp.exp(m2 - m1)
    p1 = 1.0 / (1.0 + e2)
    p2 = 1.0 - p1

    probs_ref[0] = jnp.concatenate([p1, p2], axis=1)
    idx_ref[0] = jnp.concatenate([i1, i2], axis=1)


def kernel(x, W, b):
    grid = TOKENS // BLOCK
    probs, idx = pl.pallas_call(
        _router_block,
        grid=(grid,),
        compiler_params=pltpu.CompilerParams(
            dimension_semantics=("parallel",),
        ),
        in_specs=[
            pl.BlockSpec((BLOCK, D_MODEL), lambda i: (i, 0)),
            pl.BlockSpec((D_MODEL, NUM_EXPERTS), lambda i: (0, 0)),
            pl.BlockSpec((1, NUM_EXPERTS), lambda i: (0, 0)),
        ],
        out_specs=[
            pl.BlockSpec((1, BLOCK, 2), lambda i: (i, 0, 0)),
            pl.BlockSpec((1, BLOCK, 2), lambda i: (i, 0, 0)),
        ],
        out_shape=[
            jax.ShapeDtypeStruct((grid, BLOCK, 2), jnp.float32),
            jax.ShapeDtypeStruct((grid, BLOCK, 2), jnp.int32),
        ],
    )(x, W.astype(jnp.float32), b.reshape(1, NUM_EXPERTS))
    return probs.reshape(TOKENS, 2), idx.reshape(TOKENS, 2)


# speedup vs baseline: 1.2153x; 1.0386x over previous
import jax
import jax.numpy as jnp
from jax.experimental import pallas as pl
from jax.experimental.pallas import tpu as pltpu

D_MODEL = 2048
NUM_EXPERTS = 64
TOKENS = 16384
BLOCK = 2048


def _router_block(x_ref, w_ref, b_ref, probs_ref, idx_ref):
    x = x_ref[...]
    w = w_ref[...]
    logits = jnp.dot(x, w, preferred_element_type=jnp.float32) + b_ref[...]
    iota = jax.lax.broadcasted_iota(jnp.int32, logits.shape, 1)

    m1 = jnp.max(logits, axis=1, keepdims=True)
    i1 = jnp.min(jnp.where(logits == m1, iota, NUM_EXPERTS), axis=1,
                 keepdims=True)
    masked = jnp.where(iota == i1, -jnp.inf, logits)
    m2 = jnp.max(masked, axis=1, keepdims=True)
    i2 = jnp.min(jnp.where(masked == m2, iota, NUM_EXPERTS), axis=1,
                 keepdims=True)

    e2 = jnp.exp(m2 - m1)
    p1 = 1.0 / (1.0 + e2)
    p2 = 1.0 - p1

    probs_ref[0] = jnp.concatenate([p1, p2], axis=1)
    idx_ref[0] = jnp.concatenate([i1, i2], axis=1)


def kernel(x, W, b):
    grid = TOKENS // BLOCK
    probs, idx = pl.pallas_call(
        _router_block,
        grid=(grid,),
        compiler_params=pltpu.CompilerParams(
            dimension_semantics=("parallel",),
        ),
        in_specs=[
            pl.BlockSpec((BLOCK, D_MODEL), lambda i: (i, 0)),
            pl.BlockSpec((D_MODEL, NUM_EXPERTS), lambda i: (0, 0)),
            pl.BlockSpec((1, NUM_EXPERTS), lambda i: (0, 0)),
        ],
        out_specs=[
            pl.BlockSpec((1, BLOCK, 2), lambda i: (i, 0, 0)),
            pl.BlockSpec((1, BLOCK, 2), lambda i: (i, 0, 0)),
        ],
        out_shape=[
            jax.ShapeDtypeStruct((grid, BLOCK, 2), jnp.float32),
            jax.ShapeDtypeStruct((grid, BLOCK, 2), jnp.int32),
        ],
    )(x, W.astype(jnp.float32), b.reshape(1, NUM_EXPERTS))
    return probs.reshape(TOKENS, 2), idx.reshape(TOKENS, 2)


# speedup vs baseline: 1.2170x; 1.0014x over previous
import jax
import jax.numpy as jnp
from jax.experimental import pallas as pl
from jax.experimental.pallas import tpu as pltpu

D_MODEL = 2048
NUM_EXPERTS = 64
TOKENS = 16384
BLOCK = 2048
D_HALF = D_MODEL // 2


def _router_block(xa_ref, xb_ref, w_ref, b_ref, probs_ref, idx_ref):
    w = w_ref[...]
    logits = (
        jnp.dot(xa_ref[...], w[:D_HALF], preferred_element_type=jnp.float32)
        + jnp.dot(xb_ref[...], w[D_HALF:], preferred_element_type=jnp.float32)
        + b_ref[...]
    )
    iota = jax.lax.broadcasted_iota(jnp.int32, logits.shape, 1)

    m1 = jnp.max(logits, axis=1, keepdims=True)
    i1 = jnp.min(jnp.where(logits == m1, iota, NUM_EXPERTS), axis=1,
                 keepdims=True)
    masked = jnp.where(iota == i1, -jnp.inf, logits)
    m2 = jnp.max(masked, axis=1, keepdims=True)
    i2 = jnp.min(jnp.where(masked == m2, iota, NUM_EXPERTS), axis=1,
                 keepdims=True)

    e2 = jnp.exp(m2 - m1)
    p1 = 1.0 / (1.0 + e2)
    p2 = 1.0 - p1

    probs_ref[0] = jnp.concatenate([p1, p2], axis=1)
    idx_ref[0] = jnp.concatenate([i1, i2], axis=1)


def kernel(x, W, b):
    grid = TOKENS // BLOCK
    probs, idx = pl.pallas_call(
        _router_block,
        grid=(grid,),
        compiler_params=pltpu.CompilerParams(
            dimension_semantics=("arbitrary",),
        ),
        in_specs=[
            pl.BlockSpec((BLOCK, D_HALF), lambda i: (i, 0)),
            pl.BlockSpec((BLOCK, D_HALF), lambda i: (i, 1)),
            pl.BlockSpec((D_MODEL, NUM_EXPERTS), lambda i: (0, 0)),
            pl.BlockSpec((1, NUM_EXPERTS), lambda i: (0, 0)),
        ],
        out_specs=[
            pl.BlockSpec((1, BLOCK, 2), lambda i: (i, 0, 0)),
            pl.BlockSpec((1, BLOCK, 2), lambda i: (i, 0, 0)),
        ],
        out_shape=[
            jax.ShapeDtypeStruct((grid, BLOCK, 2), jnp.float32),
            jax.ShapeDtypeStruct((grid, BLOCK, 2), jnp.int32),
        ],
    )(x, x, W.astype(jnp.float32), b.reshape(1, NUM_EXPERTS))
    return probs.reshape(TOKENS, 2), idx.reshape(TOKENS, 2)
